# Initial kernel scaffold; baseline (speedup 1.0000x reference)
import jax
import jax.numpy as jnp
from jax.experimental import pallas as pl


def kernel(x, edge_index, W1, b1, W2, b2):
    raise NotImplementedError("write your pallas kernel here")


# speedup vs baseline: 33.6537x; 33.6537x over previous
import functools

import jax
import jax.numpy as jnp
from jax import lax
from jax.experimental import pallas as pl
from jax.experimental.pallas import tpu as pltpu
from jax.experimental.pallas import tpu_sc as plsc

N = 10000
E = 320000
D_IN = 128
H = 32
W2P = 16

NC, NS = 2, 16
NT = NC * NS
CHUNK = 128
NCHUNK = 80
PER_TILE = NCHUNK * CHUNK
EPAD = NT * PER_TILE
NP = 10240
ROWS_PER_TILE = NP // NS

_mesh = plsc.VectorSubcoreMesh(core_axis_name="c", subcore_axis_name="s")


def _make_edge_pass(width):
  @functools.partial(
      pl.kernel,
      out_type=jax.ShapeDtypeStruct((NC, NP, width), jnp.float32),
      mesh=_mesh,
      compiler_params=pltpu.CompilerParams(use_tc_tiling_on_sc=False),
      scratch_types=[
          pltpu.VMEM((NCHUNK, CHUNK), jnp.int32),
          pltpu.VMEM((NCHUNK, CHUNK), jnp.int32),
          pltpu.VMEM((2, CHUNK, width), jnp.float32),
          pltpu.VMEM_SHARED((NP, width), jnp.float32),
          pltpu.SemaphoreType.DMA,
          pltpu.SemaphoreType.DMA,
      ],
  )
  def edge_pass(src_hbm, dst_hbm, y_hbm, out_hbm,
                src_v, dst_v, vals_v, agg_s, sem0, sem1):
    c = lax.axis_index("c")
    s = lax.axis_index("s")
    wid = c * NS + s

    @pl.loop(0, CHUNK)
    def _(r):
      for k in range(width // 16):
        vals_v.at[0, r, pl.ds(k * 16, 16)][...] = jnp.zeros((16,), jnp.float32)

    row0 = s * ROWS_PER_TILE
    @pl.loop(0, ROWS_PER_TILE // CHUNK)
    def _(i):
      pltpu.sync_copy(vals_v.at[0], agg_s.at[pl.ds(row0 + i * CHUNK, CHUNK)])

    pltpu.sync_copy(src_hbm.at[wid], src_v)
    pltpu.sync_copy(dst_hbm.at[wid], dst_v)

    plsc.subcore_barrier()

    def start(j, b, sem):
      pltpu.async_copy(y_hbm.at[src_v.at[j]], vals_v.at[b], sem)

    def finish(j, b, sem):
      pltpu.make_async_copy(y_hbm.at[src_v.at[j]], vals_v.at[b], sem).wait()
      pltpu.sync_copy(vals_v.at[b], agg_s.at[dst_v.at[j]], add=True)

    start(0, 0, sem0)
    start(1, 1, sem1)

    @pl.loop(0, NCHUNK - 2, step=2)
    def _(j):
      finish(j, 0, sem0)
      start(j + 2, 0, sem0)
      finish(j + 1, 1, sem1)
      start(j + 3, 1, sem1)

    finish(NCHUNK - 2, 0, sem0)
    finish(NCHUNK - 1, 1, sem1)

    plsc.subcore_barrier()

    @pl.loop(0, ROWS_PER_TILE // CHUNK)
    def _(i):
      r = row0 + i * CHUNK
      pltpu.sync_copy(agg_s.at[pl.ds(r, CHUNK)],
                      out_hbm.at[c].at[pl.ds(r, CHUNK)])

  return edge_pass


_edge_pass_l1 = _make_edge_pass(H)
_edge_pass_l2 = _make_edge_pass(W2P)


@functools.partial(
    pl.kernel,
    out_type=jax.ShapeDtypeStruct((NC, NP), jnp.float32),
    mesh=_mesh,
    compiler_params=pltpu.CompilerParams(use_tc_tiling_on_sc=False),
    scratch_types=[
        pltpu.VMEM((NCHUNK, CHUNK), jnp.int32),
        pltpu.VMEM((CHUNK,), jnp.float32),
        pltpu.VMEM((ROWS_PER_TILE,), jnp.float32),
        pltpu.VMEM_SHARED((NP,), jnp.float32),
    ],
)
def _deg_pass(dst_hbm, out_hbm, dst_v, ones_v, zeros_v, deg_s):
  c = lax.axis_index("c")
  s = lax.axis_index("s")
  wid = c * NS + s

  for k in range(CHUNK // 16):
    ones_v.at[pl.ds(k * 16, 16)][...] = jnp.ones((16,), jnp.float32)

  @pl.loop(0, ROWS_PER_TILE // 16)
  def _(k):
    zeros_v.at[pl.ds(k * 16, 16)][...] = jnp.zeros((16,), jnp.float32)

  row0 = s * ROWS_PER_TILE
  pltpu.sync_copy(zeros_v, deg_s.at[pl.ds(row0, ROWS_PER_TILE)])
  pltpu.sync_copy(dst_hbm.at[wid], dst_v)

  plsc.subcore_barrier()

  @pl.loop(0, NCHUNK)
  def _(j):
    pltpu.sync_copy(ones_v, deg_s.at[dst_v.at[j]], add=True)

  plsc.subcore_barrier()

  pltpu.sync_copy(deg_s.at[pl.ds(row0, ROWS_PER_TILE)],
                  out_hbm.at[c].at[pl.ds(row0, ROWS_PER_TILE)])


def _tc_xw(x_ref, w_ref, o_ref):
  o_ref[...] = jnp.dot(x_ref[...], w_ref[...],
                       preferred_element_type=jnp.float32)


def _tc_scale(degp_ref, xw_ref, dis_ref, y_ref):
  deg = degp_ref[0] + degp_ref[1] + 1.0
  dis = lax.rsqrt(deg)[:, None]
  dis_ref[...] = dis
  y_ref[...] = xw_ref[...] * dis


def _tc_mid(p_ref, y_ref, dis_ref, b1_ref, w2_ref, z_ref):
  dis = dis_ref[...]
  h = dis * (p_ref[0] + p_ref[1] + y_ref[...]) + b1_ref[...]
  h = jnp.maximum(h, 0.0)
  z_ref[...] = dis * jnp.dot(h, w2_ref[...],
                             preferred_element_type=jnp.float32)


def _tc_final(q_ref, z_ref, dis_ref, b2_ref, o_ref):
  out = dis_ref[...] * (q_ref[0] + q_ref[1] + z_ref[...])
  o_ref[...] = out[:N, :2] + b2_ref[...]


def kernel(x, edge_index, W1, b1, W2, b2):
  ei = edge_index.astype(jnp.int32)
  pad = jnp.full((EPAD - E,), N, jnp.int32)
  srcp = jnp.concatenate([ei[0], pad]).reshape(NT, NCHUNK, CHUNK)
  dstp = jnp.concatenate([ei[1], pad]).reshape(NT, NCHUNK, CHUNK)
  x_pad = jnp.pad(x, ((0, NP - N), (0, 0)))
  w2_pad = jnp.pad(W2, ((0, 0), (0, W2P - 2)))

  degp = _deg_pass(dstp)

  xw = pl.pallas_call(
      _tc_xw,
      out_shape=jax.ShapeDtypeStruct((NP, H), jnp.float32),
  )(x_pad, W1)

  dis, y = pl.pallas_call(
      _tc_scale,
      out_shape=(jax.ShapeDtypeStruct((NP, 1), jnp.float32),
                 jax.ShapeDtypeStruct((NP, H), jnp.float32)),
  )(degp, xw)

  p = _edge_pass_l1(srcp, dstp, y)

  z = pl.pallas_call(
      _tc_mid,
      out_shape=jax.ShapeDtypeStruct((NP, W2P), jnp.float32),
  )(p, y, dis, b1.reshape(1, H), w2_pad)

  q = _edge_pass_l2(srcp, dstp, z)

  out = pl.pallas_call(
      _tc_final,
      out_shape=jax.ShapeDtypeStruct((N, 2), jnp.float32),
  )(q, z, dis, b2.reshape(1, 2))

  return out
